# baseline (speedup 1.0000x reference)
import jax
import jax.numpy as jnp
from jax import lax
from jax.experimental import pallas as pl
from jax.experimental.pallas import tpu as pltpu
from jax.experimental.pallas import tpu_sc as plsc

N = 10000
E = 160000
D = 256
BN = 1000
GRID = N // BN
NS = 16
EPT = E // NS
CK = 80
NG = CK // 16
CCK = 112
NGC = CCK // 16
NCH = EPT // CK
SENT = EPT
EPAD = EPT + 16
PN = 640
NP = 16
SOFF = PN
SN = PN // 128
ACCN = 768
SROW = ACCN // NS
FROW = PN // NS


def _s1_body(x_ref, w1_ref, w2_ref, a_ref, h1_ref, h2_ref, scal_ref):
    xb = x_ref[...]
    h1 = jnp.dot(xb, w1_ref[...], preferred_element_type=jnp.float32)
    h2 = jnp.dot(xb, w2_ref[...], preferred_element_type=jnp.float32)
    h1_ref[0] = h1[:, :128]
    h1_ref[1] = h1[:, 128:]
    h2_ref[0] = h2[:, :128]
    h2_ref[1] = h2[:, 128:]
    el1 = jnp.sum(h1 * a_ref[0:1], axis=1, keepdims=True)
    er1 = jnp.sum(h1 * a_ref[1:2], axis=1, keepdims=True)
    el2 = jnp.sum(h2 * a_ref[2:3], axis=1, keepdims=True)
    er2 = jnp.sum(h2 * a_ref[3:4], axis=1, keepdims=True)
    scal_ref[...] = jnp.concatenate([el1, er1, el2, er2], axis=1)


def _stage1(x, W1, W2, attn):
    return pl.pallas_call(
        _s1_body,
        grid=(GRID,),
        in_specs=[
            pl.BlockSpec((BN, D), lambda i: (i, 0)),
            pl.BlockSpec((D, D), lambda i: (0, 0)),
            pl.BlockSpec((D, D), lambda i: (0, 0)),
            pl.BlockSpec((4, D), lambda i: (0, 0)),
        ],
        out_specs=[
            pl.BlockSpec((2, BN, 128), lambda i: (0, i, 0)),
            pl.BlockSpec((2, BN, 128), lambda i: (0, i, 0)),
            pl.BlockSpec((BN, 4), lambda i: (i, 0)),
        ],
        out_shape=[
            jax.ShapeDtypeStruct((2, N, 128), jnp.float32),
            jax.ShapeDtypeStruct((2, N, 128), jnp.float32),
            jax.ShapeDtypeStruct((N, 4), jnp.float32),
        ],
    )(x, W1, W2, attn)


def _sc_body(h1_hbm, h2_hbm, scal_hbm, sd1_hbm, ew1_hbm, sd2_hbm, ew2_hbm,
             zer_hbm, zs_hbm, o1_hbm, o2_hbm, s1_hbm, s2_hbm,
             el_v, er_v, src_v, dst_v, ew_v, w_all, ex_all, eid_c,
             wbuf, hrows, rows, srows, src_w, dst_w, sidx, colbuf, acc):
    c = lax.axis_index("c")
    s = lax.axis_index("s")
    cN = c * N
    lane = lax.iota(jnp.int32, 16)
    zero16i = jnp.zeros((16,), jnp.int32)
    zero16f = jnp.zeros((16,), jnp.float32)

    pltpu.sync_copy(zs_hbm, srows)
    for g in range(NGC):
        colbuf[pl.ds(g * 16, 16)] = zero16i
    w_all[pl.ds(SENT, 16)] = zero16f
    ex_all[pl.ds(SENT, 16)] = zero16f

    for (sd_hbm, ew_hbm, h_hbm, o_hbm, s_hbm, cl, cr, rix) in (
            (sd1_hbm, ew1_hbm, h1_hbm, o1_hbm, s1_hbm, 0, 1, 0),
            (sd2_hbm, ew2_hbm, h2_hbm, o2_hbm, s2_hbm, 2, 3, 1)):
        own = c == rix
        pltpu.sync_copy(sd_hbm.at[s, 0], src_v)
        pltpu.sync_copy(sd_hbm.at[s, 1], dst_v)
        pltpu.sync_copy(ew_hbm.at[s], ew_v)
        pltpu.sync_copy(scal_hbm.at[cl], el_v)
        pltpu.sync_copy(scal_hbm.at[cr], er_v)

        def pha(j, carry):
            for g in range(NG):
                sl = pl.ds(j * CK + g * 16, 16)
                sv = src_v[sl]
                dv = dst_v[sl]
                e = plsc.load_gather(el_v, [sv]) + plsc.load_gather(er_v, [dv])
                e = jnp.where(e >= 0, e, 0.2 * e)
                ex = jnp.exp(e)
                w_all[sl] = ew_v[sl] * ex
                ex_all[sl] = ex
            return carry

        lax.fori_loop(0, NCH, pha, 0)

        def do_pass(p, pcarry):
            lo = pl.multiple_of(p * PN, PN)
            pltpu.sync_copy(zer_hbm, acc.at[pl.ds(s * SROW, SROW)])

            def compact(j, count):
                for g in range(NG):
                    sl = pl.ds(j * CK + g * 16, 16)
                    dv = dst_v[sl]
                    m = (dv >= lo) & (dv < lo + PN)
                    eidv = lane + (j * CK + g * 16)
                    plsc.store_compressed(eid_c.at[pl.ds(count, 16)], eidv, mask=m)
                    count = count + plsc.all_reduce_population_count(m)[0]
                return count

            count = lax.fori_loop(0, NCH, compact, jnp.int32(0))
            sent16 = jnp.full((16,), SENT, jnp.int32)
            for g in range(NGC):
                eid_c[pl.ds(count + g * 16, 16)] = sent16
            plsc.subcore_barrier()

            def chunk(q, carry):
                for g in range(NGC):
                    sl = pl.ds(g * 16, 16)
                    eidv = eid_c[pl.ds(q * CCK + g * 16, 16)]
                    sv = plsc.load_gather(src_v, [eidv])
                    dv = plsc.load_gather(dst_v, [eidv])
                    dn = jnp.maximum(dv - lo, 0)
                    wbuf[sl] = plsc.load_gather(w_all, [eidv])
                    src_w[sl] = sv + cN
                    dst_w[sl] = dn

                    @pl.when(own)
                    def _stage_s():
                        exv = plsc.load_gather(ex_all, [eidv])
                        erow = lane + g * 16
                        plsc.store_scatter(srows, [erow, colbuf[sl]], zero16f)
                        col = dn & 127
                        plsc.store_scatter(srows, [erow, col], exv)
                        colbuf[sl] = col
                        sidx[sl] = (dn >> 7) + SOFF

                pltpu.sync_copy(h_hbm.at[src_w], hrows)
                for g in range(NGC):
                    wv = wbuf[pl.ds(g * 16, 16)]
                    for i in range(16):
                        ei = g * 16 + i
                        w = wv[i]
                        for k in range(8):
                            sl2 = pl.ds(k * 16, 16)
                            rows[ei, sl2] = hrows[ei, sl2] * w
                pltpu.sync_copy(rows, acc.at[dst_w], add=True)

                @pl.when(own)
                def _scatter_s():
                    pltpu.sync_copy(srows, acc.at[sidx], add=True)

                return carry

            lax.fori_loop(0, (count + CCK - 1) // CCK, chunk, 0)
            plsc.subcore_barrier()
            pltpu.sync_copy(acc.at[pl.ds(s * FROW, FROW)],
                            o_hbm.at[c, pl.ds(pl.multiple_of(lo + s * FROW, 8),
                                              FROW)])

            @pl.when(own & (s == 0))
            def _flush_s():
                pltpu.sync_copy(acc.at[pl.ds(SOFF, 8)], s_hbm.at[p])

            plsc.subcore_barrier()
            return pcarry

        lax.fori_loop(0, NP, do_pass, 0)


def _stage2(h1, h2, scal_t, sd1, ew1, sd2, ew2, zer, zs):
    mesh = plsc.VectorSubcoreMesh(core_axis_name="c", subcore_axis_name="s")
    return pl.kernel(
        _sc_body,
        out_type=[
            jax.ShapeDtypeStruct((2, NP * PN, 128), jnp.float32),
            jax.ShapeDtypeStruct((2, NP * PN, 128), jnp.float32),
            jax.ShapeDtypeStruct((NP, 8, 128), jnp.float32),
            jax.ShapeDtypeStruct((NP, 8, 128), jnp.float32),
        ],
        mesh=mesh,
        compiler_params=pltpu.CompilerParams(needs_layout_passes=False),
        scratch_types=[
            pltpu.VMEM((N,), jnp.float32),
            pltpu.VMEM((N,), jnp.float32),
            pltpu.VMEM((EPAD,), jnp.int32),
            pltpu.VMEM((EPAD,), jnp.int32),
            pltpu.VMEM((EPAD,), jnp.float32),
            pltpu.VMEM((EPAD,), jnp.float32),
            pltpu.VMEM((EPAD,), jnp.float32),
            pltpu.VMEM((EPT + CCK + 16,), jnp.int32),
            pltpu.VMEM((CCK,), jnp.float32),
            pltpu.VMEM((CCK, 128), jnp.float32),
            pltpu.VMEM((CCK, 128), jnp.float32),
            pltpu.VMEM((CCK, 128), jnp.float32),
            pltpu.VMEM((CCK,), jnp.int32),
            pltpu.VMEM((CCK,), jnp.int32),
            pltpu.VMEM((CCK,), jnp.int32),
            pltpu.VMEM((CCK,), jnp.int32),
            pltpu.VMEM_SHARED((ACCN, 128), jnp.float32),
        ],
    )(h1.reshape(2 * N, 128), h2.reshape(2 * N, 128), scal_t,
      sd1, ew1, sd2, ew2, zer, zs)


def _leaky01(v):
    return jnp.where(v >= 0, v, 0.01 * v)


def _s3_body(o1a_ref, o1b_ref, o2a_ref, o2b_ref, s1_ref, s2_ref, x_ref,
             b1_ref, b2_ref, wbi_ref, bbi_ref, wsi_ref, bsi_ref, rw_ref,
             gam_ref, bet_ref, out_ref):
    inv1 = 1.0 / (s1_ref[...] + 1e-9)
    inv2 = 1.0 / (s2_ref[...] + 1e-9)
    f1a = o1a_ref[0] * inv1 + b1_ref[0:1, :128]
    f1b = o1b_ref[0] * inv1 + b1_ref[0:1, 128:]
    f2a = o2a_ref[0] * inv2 + b2_ref[0:1, :128]
    f2b = o2b_ref[0] * inv2 + b2_ref[0:1, 128:]
    f1 = jnp.concatenate([f1a, f1b], axis=1)
    f2 = jnp.concatenate([f2a, f2b], axis=1)
    dfm = f1 * f2
    ss = f1 + f2
    ia = _leaky01(jnp.dot(dfm, wbi_ref[...], preferred_element_type=jnp.float32)
                  + bbi_ref[0:1])
    ib = _leaky01(jnp.dot(ss, wsi_ref[...], preferred_element_type=jnp.float32)
                  + bsi_ref[0:1])
    alpha = 1.0 / (1.0 + jnp.exp(-rw_ref[0]))
    v = ia + ib + x_ref[...] * alpha
    mu = jnp.mean(v, axis=1, keepdims=True)
    var = jnp.mean(v * v, axis=1, keepdims=True) - mu * mu
    out_ref[...] = (gam_ref[0:1] * (v - mu) / jnp.sqrt(var + 1e-5)
                    + bet_ref[0:1])


def _stage3(o1, o2, s1, s2, x, b1, b2, Wbi, bbi, Wsi, bsi, res_w, gamma,
            beta):
    full = lambda shape: pl.BlockSpec(shape, lambda i: tuple(0 for _ in shape))
    return pl.pallas_call(
        _s3_body,
        grid=(GRID,),
        in_specs=[
            pl.BlockSpec((1, BN, 128), lambda i: (0, i, 0)),
            pl.BlockSpec((1, BN, 128), lambda i: (1, i, 0)),
            pl.BlockSpec((1, BN, 128), lambda i: (0, i, 0)),
            pl.BlockSpec((1, BN, 128), lambda i: (1, i, 0)),
            pl.BlockSpec((BN, 1), lambda i: (i, 0)),
            pl.BlockSpec((BN, 1), lambda i: (i, 0)),
            pl.BlockSpec((BN, D), lambda i: (i, 0)),
            full((1, D)), full((1, D)),
            full((D, D)), full((1, D)),
            full((D, D)), full((1, D)),
            pl.BlockSpec(memory_space=pltpu.SMEM),
            full((1, D)), full((1, D)),
        ],
        out_specs=pl.BlockSpec((BN, D), lambda i: (i, 0)),
        out_shape=jax.ShapeDtypeStruct((N, D), jnp.float32),
    )(o1, o1, o2, o2, s1, s2, x, b1, b2, Wbi, bbi, Wsi, bsi, res_w,
      gamma, beta)


def kernel(x, edge_index_rel1, edge_index_rel2, edge_weight_rel1,
           edge_weight_rel2, W1, al1, ar1, b1, W2, al2, ar2, b2,
           Wbi, bbi, Wsi, bsi, res_w, gamma, beta):
    attn = jnp.concatenate([al1, ar1, al2, ar2], axis=0)
    h1, h2, scal = _stage1(x, W1, W2, attn)

    zpad = jnp.zeros((2, NS, EPAD - EPT), jnp.int32)
    sd1 = jnp.concatenate([edge_index_rel1.reshape(2, NS, EPT), zpad],
                          axis=2).transpose(1, 0, 2)
    sd2 = jnp.concatenate([edge_index_rel2.reshape(2, NS, EPT), zpad],
                          axis=2).transpose(1, 0, 2)
    wpad = jnp.zeros((NS, EPAD - EPT), jnp.float32)
    ewr1 = jnp.concatenate([edge_weight_rel1.reshape(NS, EPT), wpad], axis=1)
    ewr2 = jnp.concatenate([edge_weight_rel2.reshape(NS, EPT), wpad], axis=1)
    zer = jnp.zeros((SROW, 128), jnp.float32)
    zs = jnp.zeros((CCK, 128), jnp.float32)
    o1, o2, so1, so2 = _stage2(h1, h2, scal.T, sd1, ewr1, sd2, ewr2, zer, zs)
    s1 = so1[:, :SN].reshape(NP * PN, 1)[:N]
    s2 = so2[:, :SN].reshape(NP * PN, 1)[:N]
    return _stage3(o1, o2, s1, s2, x, b1, b2, Wbi.reshape(D, D),
                   bbi.reshape(1, D), Wsi.reshape(D, D), bsi.reshape(1, D),
                   res_w, gamma.reshape(1, D), beta.reshape(1, D))


# speedup vs baseline: 1.3365x; 1.3365x over previous
import jax
import jax.numpy as jnp
from jax import lax
from jax.experimental import pallas as pl
from jax.experimental.pallas import tpu as pltpu
from jax.experimental.pallas import tpu_sc as plsc

N = 10000
E = 160000
D = 256
BN = 1000
GRID = N // BN
NS = 16
EPT = E // NS
CK = 80
NG = CK // 16
NCH = EPT // CK
SENT = EPT
EPAD = EPT + 16
PN = 1024
NP = 10240 // PN
SOFF = PN
SN = PN // 128
ACCN = 1152
SROW = ACCN // NS
FROW = PN // NS


def _s1_body(x_ref, w1_ref, w2_ref, a_ref, h1_ref, h2_ref, scal_ref):
    xb = x_ref[...]
    h1 = jnp.dot(xb, w1_ref[...], preferred_element_type=jnp.float32)
    h2 = jnp.dot(xb, w2_ref[...], preferred_element_type=jnp.float32)
    h1_ref[0] = h1[:, :128]
    h1_ref[1] = h1[:, 128:]
    h2_ref[0] = h2[:, :128]
    h2_ref[1] = h2[:, 128:]
    el1 = jnp.sum(h1 * a_ref[0:1], axis=1, keepdims=True)
    er1 = jnp.sum(h1 * a_ref[1:2], axis=1, keepdims=True)
    el2 = jnp.sum(h2 * a_ref[2:3], axis=1, keepdims=True)
    er2 = jnp.sum(h2 * a_ref[3:4], axis=1, keepdims=True)
    scal_ref[...] = jnp.concatenate([el1, er1, el2, er2], axis=1)


def _stage1(x, W1, W2, attn):
    return pl.pallas_call(
        _s1_body,
        grid=(GRID,),
        in_specs=[
            pl.BlockSpec((BN, D), lambda i: (i, 0)),
            pl.BlockSpec((D, D), lambda i: (0, 0)),
            pl.BlockSpec((D, D), lambda i: (0, 0)),
            pl.BlockSpec((4, D), lambda i: (0, 0)),
        ],
        out_specs=[
            pl.BlockSpec((2, BN, 128), lambda i: (0, i, 0)),
            pl.BlockSpec((2, BN, 128), lambda i: (0, i, 0)),
            pl.BlockSpec((BN, 4), lambda i: (i, 0)),
        ],
        out_shape=[
            jax.ShapeDtypeStruct((2, N, 128), jnp.float32),
            jax.ShapeDtypeStruct((2, N, 128), jnp.float32),
            jax.ShapeDtypeStruct((N, 4), jnp.float32),
        ],
    )(x, W1, W2, attn)


def _sc_body(h1_hbm, h2_hbm, scal_hbm, sd1_hbm, ew1_hbm, sd2_hbm, ew2_hbm,
             zer_hbm, zs_hbm, o1_hbm, o2_hbm, s1_hbm, s2_hbm,
             el_v, er_v, src_v, dst_v, ew_v, w_all, ex_all, eid_c,
             wbuf, hrows, rows, srows, src_w, dst_w, sidx, colbuf, acc):
    c = lax.axis_index("c")
    s = lax.axis_index("s")
    cN = c * N
    lane = lax.iota(jnp.int32, 16)
    zero16i = jnp.zeros((16,), jnp.int32)
    zero16f = jnp.zeros((16,), jnp.float32)

    pltpu.sync_copy(zs_hbm, srows)
    for g in range(NG):
        colbuf[pl.ds(g * 16, 16)] = zero16i
    w_all[pl.ds(SENT, 16)] = zero16f
    ex_all[pl.ds(SENT, 16)] = zero16f

    for (sd_hbm, ew_hbm, h_hbm, o_hbm, s_hbm, cl, cr, rix) in (
            (sd1_hbm, ew1_hbm, h1_hbm, o1_hbm, s1_hbm, 0, 1, 0),
            (sd2_hbm, ew2_hbm, h2_hbm, o2_hbm, s2_hbm, 2, 3, 1)):
        own = c == rix
        pltpu.sync_copy(sd_hbm.at[s, 0], src_v)
        pltpu.sync_copy(sd_hbm.at[s, 1], dst_v)
        pltpu.sync_copy(ew_hbm.at[s], ew_v)
        pltpu.sync_copy(scal_hbm.at[cl], el_v)
        pltpu.sync_copy(scal_hbm.at[cr], er_v)

        def pha(j, carry):
            for g in range(NG):
                sl = pl.ds(j * CK + g * 16, 16)
                sv = src_v[sl]
                dv = dst_v[sl]
                e = plsc.load_gather(el_v, [sv]) + plsc.load_gather(er_v, [dv])
                e = jnp.where(e >= 0, e, 0.2 * e)
                ex = jnp.exp(e)
                w_all[sl] = ew_v[sl] * ex
                ex_all[sl] = ex
            return carry

        lax.fori_loop(0, NCH, pha, 0)

        def do_pass(p, pcarry):
            lo = pl.multiple_of(p * PN, PN)
            pltpu.sync_copy(zer_hbm, acc.at[pl.ds(s * SROW, SROW)])

            def compact(j, count):
                for g in range(NG):
                    sl = pl.ds(j * CK + g * 16, 16)
                    dv = dst_v[sl]
                    m = (dv >= lo) & (dv < lo + PN)
                    eidv = lane + (j * CK + g * 16)
                    plsc.store_compressed(eid_c.at[pl.ds(count, 16)], eidv, mask=m)
                    count = count + plsc.all_reduce_population_count(m)[0]
                return count

            count = lax.fori_loop(0, NCH, compact, jnp.int32(0))
            sent16 = jnp.full((16,), SENT, jnp.int32)
            for g in range(NG):
                eid_c[pl.ds(count + g * 16, 16)] = sent16
            plsc.subcore_barrier()

            def chunk(q, carry):
                for g in range(NG):
                    sl = pl.ds(g * 16, 16)
                    eidv = eid_c[pl.ds(q * CK + g * 16, 16)]
                    sv = plsc.load_gather(src_v, [eidv])
                    dv = plsc.load_gather(dst_v, [eidv])
                    dn = jnp.maximum(dv - lo, 0)
                    wbuf[sl] = plsc.load_gather(w_all, [eidv])
                    src_w[sl] = sv + cN
                    dst_w[sl] = dn

                    @pl.when(own)
                    def _stage_s():
                        exv = plsc.load_gather(ex_all, [eidv])
                        erow = lane + g * 16
                        plsc.store_scatter(srows, [erow, colbuf[sl]], zero16f)
                        col = dn & 127
                        plsc.store_scatter(srows, [erow, col], exv)
                        colbuf[sl] = col
                        sidx[sl] = (dn >> 7) + SOFF

                pltpu.sync_copy(h_hbm.at[src_w], hrows)
                for g in range(NG):
                    wv = wbuf[pl.ds(g * 16, 16)]
                    for i in range(16):
                        ei = g * 16 + i
                        w = wv[i]
                        for k in range(8):
                            sl2 = pl.ds(k * 16, 16)
                            rows[ei, sl2] = hrows[ei, sl2] * w
                pltpu.sync_copy(rows, acc.at[dst_w], add=True)

                @pl.when(own)
                def _scatter_s():
                    pltpu.sync_copy(srows, acc.at[sidx], add=True)

                return carry

            lax.fori_loop(0, (count + CK - 1) // CK, chunk, 0)
            plsc.subcore_barrier()
            pltpu.sync_copy(acc.at[pl.ds(s * FROW, FROW)],
                            o_hbm.at[c, pl.ds(pl.multiple_of(lo + s * FROW, 8),
                                              FROW)])

            @pl.when(own & (s == 0))
            def _flush_s():
                pltpu.sync_copy(acc.at[pl.ds(SOFF, SN)],
                                s_hbm.at[pl.ds(pl.multiple_of(p * SN, 8), SN)])

            plsc.subcore_barrier()
            return pcarry

        lax.fori_loop(0, NP, do_pass, 0)


def _stage2(h1, h2, scal_t, sd1, ew1, sd2, ew2, zer, zs):
    mesh = plsc.VectorSubcoreMesh(core_axis_name="c", subcore_axis_name="s")
    return pl.kernel(
        _sc_body,
        out_type=[
            jax.ShapeDtypeStruct((2, NP * PN, 128), jnp.float32),
            jax.ShapeDtypeStruct((2, NP * PN, 128), jnp.float32),
            jax.ShapeDtypeStruct((NP * SN, 128), jnp.float32),
            jax.ShapeDtypeStruct((NP * SN, 128), jnp.float32),
        ],
        mesh=mesh,
        compiler_params=pltpu.CompilerParams(needs_layout_passes=False),
        scratch_types=[
            pltpu.VMEM((N,), jnp.float32),
            pltpu.VMEM((N,), jnp.float32),
            pltpu.VMEM((EPAD,), jnp.int32),
            pltpu.VMEM((EPAD,), jnp.int32),
            pltpu.VMEM((EPAD,), jnp.float32),
            pltpu.VMEM((EPAD,), jnp.float32),
            pltpu.VMEM((EPAD,), jnp.float32),
            pltpu.VMEM((EPT + CK,), jnp.int32),
            pltpu.VMEM((CK,), jnp.float32),
            pltpu.VMEM((CK, 128), jnp.float32),
            pltpu.VMEM((CK, 128), jnp.float32),
            pltpu.VMEM((CK, 128), jnp.float32),
            pltpu.VMEM((CK,), jnp.int32),
            pltpu.VMEM((CK,), jnp.int32),
            pltpu.VMEM((CK,), jnp.int32),
            pltpu.VMEM((CK,), jnp.int32),
            pltpu.VMEM_SHARED((ACCN, 128), jnp.float32),
        ],
    )(h1.reshape(2 * N, 128), h2.reshape(2 * N, 128), scal_t,
      sd1, ew1, sd2, ew2, zer, zs)


def _leaky01(v):
    return jnp.where(v >= 0, v, 0.01 * v)


def _s3_body(o1a_ref, o1b_ref, o2a_ref, o2b_ref, s1_ref, s2_ref, x_ref,
             b1_ref, b2_ref, wbi_ref, bbi_ref, wsi_ref, bsi_ref, rw_ref,
             gam_ref, bet_ref, out_ref):
    inv1 = 1.0 / (s1_ref[...] + 1e-9)
    inv2 = 1.0 / (s2_ref[...] + 1e-9)
    f1a = o1a_ref[0] * inv1 + b1_ref[0:1, :128]
    f1b = o1b_ref[0] * inv1 + b1_ref[0:1, 128:]
    f2a = o2a_ref[0] * inv2 + b2_ref[0:1, :128]
    f2b = o2b_ref[0] * inv2 + b2_ref[0:1, 128:]
    f1 = jnp.concatenate([f1a, f1b], axis=1)
    f2 = jnp.concatenate([f2a, f2b], axis=1)
    dfm = f1 * f2
    ss = f1 + f2
    ia = _leaky01(jnp.dot(dfm, wbi_ref[...], preferred_element_type=jnp.float32)
                  + bbi_ref[0:1])
    ib = _leaky01(jnp.dot(ss, wsi_ref[...], preferred_element_type=jnp.float32)
                  + bsi_ref[0:1])
    alpha = 1.0 / (1.0 + jnp.exp(-rw_ref[0]))
    v = ia + ib + x_ref[...] * alpha
    mu = jnp.mean(v, axis=1, keepdims=True)
    var = jnp.mean(v * v, axis=1, keepdims=True) - mu * mu
    out_ref[...] = (gam_ref[0:1] * (v - mu) / jnp.sqrt(var + 1e-5)
                    + bet_ref[0:1])


def _stage3(o1, o2, s1, s2, x, b1, b2, Wbi, bbi, Wsi, bsi, res_w, gamma,
            beta):
    full = lambda shape: pl.BlockSpec(shape, lambda i: tuple(0 for _ in shape))
    return pl.pallas_call(
        _s3_body,
        grid=(GRID,),
        in_specs=[
            pl.BlockSpec((1, BN, 128), lambda i: (0, i, 0)),
            pl.BlockSpec((1, BN, 128), lambda i: (1, i, 0)),
            pl.BlockSpec((1, BN, 128), lambda i: (0, i, 0)),
            pl.BlockSpec((1, BN, 128), lambda i: (1, i, 0)),
            pl.BlockSpec((BN, 1), lambda i: (i, 0)),
            pl.BlockSpec((BN, 1), lambda i: (i, 0)),
            pl.BlockSpec((BN, D), lambda i: (i, 0)),
            full((1, D)), full((1, D)),
            full((D, D)), full((1, D)),
            full((D, D)), full((1, D)),
            pl.BlockSpec(memory_space=pltpu.SMEM),
            full((1, D)), full((1, D)),
        ],
        out_specs=pl.BlockSpec((BN, D), lambda i: (i, 0)),
        out_shape=jax.ShapeDtypeStruct((N, D), jnp.float32),
    )(o1, o1, o2, o2, s1, s2, x, b1, b2, Wbi, bbi, Wsi, bsi, res_w,
      gamma, beta)


def kernel(x, edge_index_rel1, edge_index_rel2, edge_weight_rel1,
           edge_weight_rel2, W1, al1, ar1, b1, W2, al2, ar2, b2,
           Wbi, bbi, Wsi, bsi, res_w, gamma, beta):
    attn = jnp.concatenate([al1, ar1, al2, ar2], axis=0)
    h1, h2, scal = _stage1(x, W1, W2, attn)

    zpad = jnp.zeros((2, NS, EPAD - EPT), jnp.int32)
    sd1 = jnp.concatenate([edge_index_rel1.reshape(2, NS, EPT), zpad],
                          axis=2).transpose(1, 0, 2)
    sd2 = jnp.concatenate([edge_index_rel2.reshape(2, NS, EPT), zpad],
                          axis=2).transpose(1, 0, 2)
    wpad = jnp.zeros((NS, EPAD - EPT), jnp.float32)
    ewr1 = jnp.concatenate([edge_weight_rel1.reshape(NS, EPT), wpad], axis=1)
    ewr2 = jnp.concatenate([edge_weight_rel2.reshape(NS, EPT), wpad], axis=1)
    zer = jnp.zeros((SROW, 128), jnp.float32)
    zs = jnp.zeros((CK, 128), jnp.float32)
    o1, o2, so1, so2 = _stage2(h1, h2, scal.T, sd1, ewr1, sd2, ewr2, zer, zs)
    s1 = so1.reshape(NP * PN, 1)[:N]
    s2 = so2.reshape(NP * PN, 1)[:N]
    return _stage3(o1, o2, s1, s2, x, b1, b2, Wbi.reshape(D, D),
                   bbi.reshape(1, D), Wsi.reshape(D, D), bsi.reshape(1, D),
                   res_w, gamma.reshape(1, D), beta.reshape(1, D))
